# speedup vs baseline: 1.2416x; 1.2416x over previous
import functools

import jax
import jax.numpy as jnp
from jax import lax
from jax.experimental import pallas as pl
from jax.experimental.pallas import tpu as pltpu
from jax.experimental.pallas import tpu_sc as plsc

_B, _CIN, _COUT, _D = 64, 256, 128, 1024
_NC, _NS, _L = 2, 16, 16
_NW = _NC * _NS
_R = _B * _COUT
_RPW = _R // _NW
_BPW = _B // _NW
_CH = 64
_NCHUNK = _RPW // _CH

_mesh = plsc.VectorSubcoreMesh(core_axis_name="c", subcore_axis_name="s")


@functools.partial(
    pl.kernel,
    mesh=_mesh,
    out_type=jax.ShapeDtypeStruct((_R, _D), jnp.float32),
    scratch_types=[
        pltpu.VMEM((_COUT,), jnp.int32),
        pltpu.VMEM((_RPW,), jnp.int32),
        pltpu.VMEM((_CH, _D), jnp.float32),
        pltpu.SemaphoreType.DMA,
    ],
)
def _gather(x_hbm, idx_hbm, out_hbm, tab_v, rows_idx_v, buf_v, sem):
    wid = lax.axis_index("s") * _NC + lax.axis_index("c")
    base = wid * _RPW
    pltpu.sync_copy(idx_hbm, tab_v)
    for k in range(_RPW // _L):
        j0 = (k * _L) % _COUT
        b = wid * _BPW + (k * _L) // _COUT
        rows_idx_v[pl.ds(k * _L, _L)] = tab_v[pl.ds(j0, _L)] + b * _CIN
    for ci in range(_NCHUNK):
        off = ci * _CH
        pltpu.async_copy(
            x_hbm.at[rows_idx_v.at[pl.ds(off, _CH)]], buf_v, sem
        ).wait()
        pltpu.sync_copy(buf_v, out_hbm.at[pl.ds(base + off, _CH)])


def kernel(x, indices):
    out = _gather(x.reshape(_B * _CIN, _D), indices)
    return out.reshape(_B, _COUT, _D)


# speedup vs baseline: 1.2659x; 1.0195x over previous
import functools

import jax
import jax.numpy as jnp
from jax import lax
from jax.experimental import pallas as pl
from jax.experimental.pallas import tpu as pltpu
from jax.experimental.pallas import tpu_sc as plsc

_B, _CIN, _COUT, _D = 64, 256, 128, 1024
_NC, _NS, _L = 2, 16, 16
_NW = _NC * _NS
_R = _B * _COUT
_RPW = _R // _NW
_BPW = _B // _NW
_CH = 32
_NCHUNK = _RPW // _CH
_NBUF = 3

_mesh = plsc.VectorSubcoreMesh(core_axis_name="c", subcore_axis_name="s")


@functools.partial(
    pl.kernel,
    mesh=_mesh,
    out_type=jax.ShapeDtypeStruct((_R, _D), jnp.float32),
    scratch_types=[
        pltpu.VMEM((_COUT,), jnp.int32),
        pltpu.VMEM((_RPW,), jnp.int32),
    ]
    + [pltpu.VMEM((_CH, _D), jnp.float32) for _ in range(_NBUF)]
    + [pltpu.SemaphoreType.DMA for _ in range(2 * _NBUF)],
)
def _gather(x_hbm, idx_hbm, out_hbm, tab_v, rows_idx_v, *bufs_and_sems):
    bufs = bufs_and_sems[:_NBUF]
    gsems = bufs_and_sems[_NBUF:2 * _NBUF]
    ssems = bufs_and_sems[2 * _NBUF:]
    wid = lax.axis_index("s") * _NC + lax.axis_index("c")
    base = wid * _RPW
    pltpu.sync_copy(idx_hbm, tab_v)
    for k in range(_RPW // _L):
        j0 = (k * _L) % _COUT
        b = wid * _BPW + (k * _L) // _COUT
        rows_idx_v[pl.ds(k * _L, _L)] = tab_v[pl.ds(j0, _L)] + b * _CIN
    gh = [None] * _NCHUNK
    sh = [None] * _NCHUNK
    for ci in range(_NCHUNK):
        p = ci % _NBUF
        if ci >= _NBUF:
            sh[ci - _NBUF].wait()
        gh[ci] = pltpu.async_copy(
            x_hbm.at[rows_idx_v.at[pl.ds(ci * _CH, _CH)]], bufs[p], gsems[p]
        )
        if ci >= 1:
            q = (ci - 1) % _NBUF
            gh[ci - 1].wait()
            sh[ci - 1] = pltpu.async_copy(
                bufs[q], out_hbm.at[pl.ds(base + (ci - 1) * _CH, _CH)],
                ssems[q],
            )
    last = _NCHUNK - 1
    gh[last].wait()
    sh[last] = pltpu.async_copy(
        bufs[last % _NBUF], out_hbm.at[pl.ds(base + last * _CH, _CH)],
        ssems[last % _NBUF],
    )
    for ci in range(max(0, _NCHUNK - _NBUF), _NCHUNK):
        sh[ci].wait()


def kernel(x, indices):
    out = _gather(x.reshape(_B * _CIN, _D), indices)
    return out.reshape(_B, _COUT, _D)


# speedup vs baseline: 1.3020x; 1.0286x over previous
import functools

import jax
import jax.numpy as jnp
from jax import lax
from jax.experimental import pallas as pl
from jax.experimental.pallas import tpu as pltpu
from jax.experimental.pallas import tpu_sc as plsc

_B, _CIN, _COUT, _D = 64, 256, 128, 1024
_NC, _NS, _L = 2, 16, 16
_NW = _NC * _NS
_R = _B * _COUT
_RPW = _R // _NW
_BPW = _B // _NW
_CH = 16
_NCHUNK = _RPW // _CH
_NBUF = 6

_mesh = plsc.VectorSubcoreMesh(core_axis_name="c", subcore_axis_name="s")


@functools.partial(
    pl.kernel,
    mesh=_mesh,
    out_type=jax.ShapeDtypeStruct((_R, _D), jnp.float32),
    scratch_types=[
        pltpu.VMEM((_COUT,), jnp.int32),
        pltpu.VMEM((_RPW,), jnp.int32),
    ]
    + [pltpu.VMEM((_CH, _D), jnp.float32) for _ in range(_NBUF)]
    + [pltpu.SemaphoreType.DMA for _ in range(2 * _NBUF)],
)
def _gather(x_hbm, idx_hbm, out_hbm, tab_v, rows_idx_v, *bufs_and_sems):
    bufs = bufs_and_sems[:_NBUF]
    gsems = bufs_and_sems[_NBUF:2 * _NBUF]
    ssems = bufs_and_sems[2 * _NBUF:]
    wid = lax.axis_index("s") * _NC + lax.axis_index("c")
    base = wid * _RPW
    pltpu.sync_copy(idx_hbm, tab_v)
    for k in range(_RPW // _L):
        j0 = (k * _L) % _COUT
        b = wid * _BPW + (k * _L) // _COUT
        rows_idx_v[pl.ds(k * _L, _L)] = tab_v[pl.ds(j0, _L)] + b * _CIN
    gh = [None] * _NCHUNK
    sh = [None] * _NCHUNK
    for ci in range(_NCHUNK):
        p = ci % _NBUF
        if ci >= _NBUF:
            sh[ci - _NBUF].wait()
        gh[ci] = pltpu.async_copy(
            x_hbm.at[rows_idx_v.at[pl.ds(ci * _CH, _CH)]], bufs[p], gsems[p]
        )
        if ci >= 1:
            q = (ci - 1) % _NBUF
            gh[ci - 1].wait()
            sh[ci - 1] = pltpu.async_copy(
                bufs[q], out_hbm.at[pl.ds(base + (ci - 1) * _CH, _CH)],
                ssems[q],
            )
    last = _NCHUNK - 1
    gh[last].wait()
    sh[last] = pltpu.async_copy(
        bufs[last % _NBUF], out_hbm.at[pl.ds(base + last * _CH, _CH)],
        ssems[last % _NBUF],
    )
    for ci in range(max(0, _NCHUNK - _NBUF), _NCHUNK):
        sh[ci].wait()


def kernel(x, indices):
    out = _gather(x.reshape(_B * _CIN, _D), indices)
    return out.reshape(_B, _COUT, _D)


# speedup vs baseline: 1.3541x; 1.0400x over previous
import functools

import jax
import jax.numpy as jnp
from jax import lax
from jax.experimental import pallas as pl
from jax.experimental.pallas import tpu as pltpu
from jax.experimental.pallas import tpu_sc as plsc

_B, _CIN, _COUT, _D = 64, 256, 128, 1024
_NC, _NS, _L = 2, 16, 16
_NW = _NC * _NS
_R = _B * _COUT
_RPW = _R // _NW
_BPW = _B // _NW
_CH = 16
_NCHUNK = _RPW // _CH
_NBUF = 6

_mesh = plsc.VectorSubcoreMesh(core_axis_name="c", subcore_axis_name="s")


@functools.partial(
    pl.kernel,
    mesh=_mesh,
    out_type=jax.ShapeDtypeStruct((_R, _D), jnp.float32),
    scratch_types=[
        pltpu.VMEM((_RPW,), jnp.int32),
        pltpu.VMEM((_NBUF * _CH, _D), jnp.float32),
        pltpu.SemaphoreType.DMA((_NBUF,)),
        pltpu.SemaphoreType.DMA((_NBUF,)),
    ],
)
def _gather(x_hbm, idx_hbm, out_hbm, rows_idx_v, ring_v, gsem, ssem):
    wid = lax.axis_index("s") * _NC + lax.axis_index("c")
    base = wid * _RPW
    pltpu.sync_copy(idx_hbm, rows_idx_v.at[pl.ds(0, _COUT)])
    b_hi = (wid * _BPW + 1) * _CIN
    b_lo = (wid * _BPW) * _CIN
    for k in range(_COUT // _L):
        j0 = k * _L
        rows_idx_v[pl.ds(_COUT + j0, _L)] = rows_idx_v[pl.ds(j0, _L)] + b_hi
    for k in range(_COUT // _L):
        j0 = k * _L
        rows_idx_v[pl.ds(j0, _L)] = rows_idx_v[pl.ds(j0, _L)] + b_lo

    def _slot(ci):
        return lax.rem(ci, _NBUF)

    def _gather_start(ci):
        p = _slot(ci)
        pltpu.async_copy(
            x_hbm.at[rows_idx_v.at[pl.ds(ci * _CH, _CH)]],
            ring_v.at[pl.ds(p * _CH, _CH)],
            gsem.at[p],
        )

    def _scatter_start(ci):
        p = _slot(ci)
        pltpu.async_copy(
            ring_v.at[pl.ds(p * _CH, _CH)],
            out_hbm.at[pl.ds(base + ci * _CH, _CH)],
            ssem.at[p],
        )

    def _gather_wait(ci):
        p = _slot(ci)
        pltpu.make_async_copy(
            x_hbm.at[pl.ds(0, _CH)], ring_v.at[pl.ds(p * _CH, _CH)], gsem.at[p]
        ).wait()

    def _scatter_wait(ci):
        p = _slot(ci)
        pltpu.make_async_copy(
            x_hbm.at[pl.ds(0, _CH)],
            out_hbm.at[pl.ds(base + ci * _CH, _CH)],
            ssem.at[p],
        ).wait()

    for ci in range(_NBUF):
        _gather_start(ci)

    def body(ci, carry):
        _gather_wait(ci)
        _scatter_start(ci)

        @pl.when(ci + _NBUF < _NCHUNK)
        def _():
            _scatter_wait(ci)
            _gather_start(ci + _NBUF)

        return carry

    lax.fori_loop(0, _NCHUNK, body, 0)
    for ci in range(_NCHUNK - _NBUF, _NCHUNK):
        _scatter_wait(ci)


def kernel(x, indices):
    out = _gather(x.reshape(_B * _CIN, _D), indices)
    return out.reshape(_B, _COUT, _D)
